# Initial kernel scaffold; baseline (speedup 1.0000x reference)
import jax
import jax.numpy as jnp
from jax.experimental import pallas as pl


def kernel(x, edge_attr, edge_index, node_edge_index, face, W, b):
    raise NotImplementedError("write your pallas kernel here")


# speedup vs baseline: 7.6405x; 7.6405x over previous
import functools

import jax
import jax.numpy as jnp
from jax import lax
from jax.experimental import pallas as pl
from jax.experimental.pallas import tpu as pltpu
from jax.experimental.pallas import tpu_sc as plsc

_E = 1600000
_NGE = _E // 128
_NODES = 50000
_CELLS = 100000
_CELLSP = 100096
_NGC = _CELLSP // 128
_NC, _NS = 2, 16
_NW = _NC * _NS
_CH = 8
_NPS = _NODES // _NS
_ZR = 625


def _scatter_body(ea, nei, out0, out1, rbufA, rbufB, ibuf, zbuf, table):
    c = lax.axis_index("c")
    s = lax.axis_index("s")
    w = s * _NC + c

    def _zb(i, carry):
        zbuf[i] = jnp.zeros((16,), jnp.float32)
        return carry

    lax.fori_loop(0, _ZR, _zb, 0)
    for k in range(_NPS // _ZR):
        pltpu.sync_copy(zbuf, table.at[pl.ds(s * _NPS + k * _ZR, _ZR)])
    plsc.subcore_barrier()

    gs = w * _NGE // _NW
    ge = (w + 1) * _NGE // _NW
    nfull = (ge - gs) // _CH

    def _chunk(ci, carry):
        g0 = gs + ci * _CH
        e0 = g0 * 128
        pltpu.sync_copy(ea.at[pl.ds(e0, _CH * 128), pl.ds(0, 16)], rbufA)
        pltpu.sync_copy(ea.at[pl.ds(e0, _CH * 128), pl.ds(16, 16)], rbufB)
        pltpu.sync_copy(nei.at[0, pl.ds(g0, _CH)], ibuf.at[0])
        pltpu.sync_copy(nei.at[1, pl.ds(g0, _CH)], ibuf.at[1])
        for j in range(_CH):
            pltpu.sync_copy(rbufA.at[pl.ds(j * 128, 128)],
                            table.at[ibuf.at[0, j]], add=True)
            pltpu.sync_copy(rbufB.at[pl.ds(j * 128, 128)],
                            table.at[ibuf.at[1, j]], add=True)
        return carry

    lax.fori_loop(0, nfull, _chunk, 0)

    def _tail(g, carry):
        e0 = g * 128
        pltpu.sync_copy(ea.at[pl.ds(e0, 128), pl.ds(0, 16)],
                        rbufA.at[pl.ds(0, 128)])
        pltpu.sync_copy(ea.at[pl.ds(e0, 128), pl.ds(16, 16)],
                        rbufB.at[pl.ds(0, 128)])
        pltpu.sync_copy(nei.at[0, pl.ds(g, 1)], ibuf.at[0, pl.ds(0, 1)])
        pltpu.sync_copy(nei.at[1, pl.ds(g, 1)], ibuf.at[1, pl.ds(0, 1)])
        pltpu.sync_copy(rbufA.at[pl.ds(0, 128)], table.at[ibuf.at[0, 0]],
                        add=True)
        pltpu.sync_copy(rbufB.at[pl.ds(0, 128)], table.at[ibuf.at[1, 0]],
                        add=True)
        return carry

    lax.fori_loop(gs + nfull * _CH, ge, _tail, 0)
    plsc.subcore_barrier()

    @pl.when(c == 0)
    def _():
        pltpu.sync_copy(table.at[pl.ds(s * _NPS, _NPS)],
                        out0.at[pl.ds(s * _NPS, _NPS)])

    @pl.when(c == 1)
    def _():
        pltpu.sync_copy(table.at[pl.ds(s * _NPS, _NPS)],
                        out1.at[pl.ds(s * _NPS, _NPS)])


def _gather_body(p0, p1, fc, cells, ibuf, rbuf, obuf, sem):
    c = lax.axis_index("c")
    s = lax.axis_index("s")
    w = s * _NC + c
    gs = w * _NGC // _NW
    ge = (w + 1) * _NGC // _NW

    def _grp(g, carry):
        for j in range(3):
            pltpu.sync_copy(fc.at[j, g], ibuf.at[j])
        cps = []
        for j in range(3):
            cps.append(pltpu.async_copy(p0.at[ibuf.at[j]], rbuf.at[j], sem))
            cps.append(pltpu.async_copy(p1.at[ibuf.at[j]], rbuf.at[3 + j], sem))
        for cp in cps:
            cp.wait()

        def _cell(i, cc):
            acc = ((rbuf[0, i] + rbuf[1, i]) + (rbuf[2, i] + rbuf[3, i])
                   + (rbuf[4, i] + rbuf[5, i]))
            obuf[i] = acc
            return cc

        lax.fori_loop(0, 128, _cell, 0)
        pltpu.sync_copy(obuf, cells.at[pl.ds(g * 128, 128)])
        return carry

    lax.fori_loop(gs, ge, _grp, 0)


def _mm_body(x_ref, cl_ref, wx_ref, wc_ref, b_ref, o_ref):
    o_ref[...] = (jnp.dot(x_ref[...], wx_ref[...],
                          preferred_element_type=jnp.float32)
                  + jnp.dot(cl_ref[...], wc_ref[...],
                            preferred_element_type=jnp.float32)
                  + b_ref[...])


def _make_sc_kernels():
    mesh = plsc.VectorSubcoreMesh(core_axis_name="c", subcore_axis_name="s",
                                  num_cores=_NC, num_subcores=_NS)
    params = pltpu.CompilerParams(use_tc_tiling_on_sc=False)
    scatter = pl.kernel(
        _scatter_body,
        compiler_params=params,
        out_type=(jax.ShapeDtypeStruct((_NODES, 16), jnp.float32),
                  jax.ShapeDtypeStruct((_NODES, 16), jnp.float32)),
        mesh=mesh,
        scratch_types=[
            pltpu.VMEM((_CH * 128, 16), jnp.float32),
            pltpu.VMEM((_CH * 128, 16), jnp.float32),
            pltpu.VMEM((2, _CH, 128), jnp.int32),
            pltpu.VMEM((_ZR, 16), jnp.float32),
            pltpu.VMEM_SHARED((_NODES, 16), jnp.float32),
        ],
    )
    gather = pl.kernel(
        _gather_body,
        compiler_params=params,
        out_type=jax.ShapeDtypeStruct((_CELLSP, 16), jnp.float32),
        mesh=mesh,
        scratch_types=[
            pltpu.VMEM((3, 128), jnp.int32),
            pltpu.VMEM((6, 128, 16), jnp.float32),
            pltpu.VMEM((128, 16), jnp.float32),
            pltpu.SemaphoreType.DMA,
        ],
    )
    return scatter, gather


def _matmul(x, cl, wx, wc, b2):
    blk = 2000
    return pl.pallas_call(
        _mm_body,
        grid=(_CELLS // blk,),
        in_specs=[
            pl.BlockSpec((blk, 128), lambda i: (i, 0)),
            pl.BlockSpec((blk, 16), lambda i: (i, 0)),
            pl.BlockSpec((128, 128), lambda i: (0, 0)),
            pl.BlockSpec((16, 128), lambda i: (0, 0)),
            pl.BlockSpec((1, 128), lambda i: (0, 0)),
        ],
        out_specs=pl.BlockSpec((blk, 128), lambda i: (i, 0)),
        out_shape=jax.ShapeDtypeStruct((_CELLS, 128), jnp.float32),
    )(x, cl, wx, wc, b2)


def kernel(x, edge_attr, edge_index, node_edge_index, face, W, b):
    nei = node_edge_index.astype(jnp.int32).reshape(2, _NGE, 128)
    fcp = jnp.pad(face.astype(jnp.int32),
                  ((0, 0), (0, _CELLSP - _CELLS))).reshape(3, _NGC, 128)
    scatter, gather = _make_sc_kernels()
    p0, p1 = scatter(edge_attr, nei)
    cells = gather(p0, p1, fcp)
    wx = W[:128]
    wc = W[128:] * (1.0 / 3.0)
    out = _matmul(x, cells, wx, wc, b.reshape(1, 128))
    return (out, edge_attr, edge_index)
